# baseline (speedup 1.0000x reference)
import functools

import jax
import jax.numpy as jnp
from jax import lax
from jax.experimental import pallas as pl
from jax.experimental.pallas import tpu as pltpu
from jax.experimental.pallas import tpu_sc as plsc

_NC, _NS, _L = 2, 16, 16
_NW = _NC * _NS
_R = 16
_U = 8


def kernel(x, embed_weight):
    B, T, E = x.shape
    rows_w = T // _NW
    n_chunks = rows_w // _R
    CH = _R * E

    mesh = plsc.VectorSubcoreMesh(core_axis_name="c", subcore_axis_name="s")

    @functools.partial(
        pl.kernel,
        mesh=mesh,
        out_type=jax.ShapeDtypeStruct((B * B * T * E,), jnp.float32),
        scratch_types=[
            pltpu.VMEM((CH,), jnp.float32),
            pltpu.VMEM((CH,), jnp.float32),
            pltpu.VMEM((CH,), jnp.float32),
            pltpu.SemaphoreType.DMA,
        ],
    )
    def sc_add(x_hbm, w_hbm, out_hbm, b0, b1, bw, sem):
        wid = lax.axis_index("s") * _NC + lax.axis_index("c")
        row0 = wid * rows_w

        def chunk_body(c, carry):
            base = (row0 + c * _R) * E
            c0 = pltpu.async_copy(x_hbm.at[pl.ds(base, CH)], b0, sem)
            c1 = pltpu.async_copy(x_hbm.at[pl.ds(T * E + base, CH)], b1, sem)
            c2 = pltpu.async_copy(w_hbm.at[pl.ds(base, CH)], bw, sem)
            c0.wait()
            c1.wait()
            c2.wait()

            def vec_body(i, carry2):
                off = i * (_L * _U)
                for u in range(_U):
                    sl = pl.ds(off + u * _L, _L)
                    wv = bw[sl]
                    b0[sl] = b0[sl] + wv
                    b1[sl] = b1[sl] + wv
                return carry2

            lax.fori_loop(0, CH // (_L * _U), vec_body, 0)

            outs = []
            for i in range(B):
                for j in range(B):
                    dst = out_hbm.at[pl.ds(((i * B + j) * T) * E + base, CH)]
                    outs.append(pltpu.async_copy(b0 if j == 0 else b1, dst, sem))
            for o in outs:
                o.wait()
            return carry

        lax.fori_loop(0, n_chunks, chunk_body, 0)

    out = sc_add(x.reshape(-1), embed_weight.reshape(-1))
    return out.reshape(B, B, T, E)


# speedup vs baseline: 1.0060x; 1.0060x over previous
import functools

import jax
import jax.numpy as jnp
from jax import lax
from jax.experimental import pallas as pl
from jax.experimental.pallas import tpu as pltpu
from jax.experimental.pallas import tpu_sc as plsc

_NC, _NS, _L = 2, 16, 16
_NW = _NC * _NS
_R = 16
_U = 8


def kernel(x, embed_weight):
    B, T, E = x.shape
    rows_w = T // _NW
    n_chunks = rows_w // _R
    CH = _R * E

    mesh = plsc.VectorSubcoreMesh(core_axis_name="c", subcore_axis_name="s")

    @functools.partial(
        pl.kernel,
        mesh=mesh,
        out_type=jax.ShapeDtypeStruct((B * B * T * E,), jnp.float32),
        scratch_types=[
            pltpu.VMEM((2, CH), jnp.float32),
            pltpu.VMEM((2, CH), jnp.float32),
            pltpu.VMEM((2, CH), jnp.float32),
            pltpu.SemaphoreType.DMA,
            pltpu.SemaphoreType.DMA,
            pltpu.SemaphoreType.DMA,
            pltpu.SemaphoreType.DMA,
        ],
    )
    def sc_add(x_hbm, w_hbm, out_hbm, b0, b1, bw, si0, si1, so0, so1):
        wid = lax.axis_index("s") * _NC + lax.axis_index("c")
        row0 = wid * rows_w
        sem_in = (si0, si1)
        sem_out = (so0, so1)

        def start_in(c):
            s = c % 2
            base = (row0 + c * _R) * E
            return [
                pltpu.async_copy(x_hbm.at[pl.ds(base, CH)], b0.at[s], sem_in[s]),
                pltpu.async_copy(x_hbm.at[pl.ds(T * E + base, CH)], b1.at[s],
                                 sem_in[s]),
                pltpu.async_copy(w_hbm.at[pl.ds(base, CH)], bw.at[s], sem_in[s]),
            ]

        def start_out(c):
            s = c % 2
            base = (row0 + c * _R) * E
            cps = []
            for i in range(B):
                for j in range(B):
                    dst = out_hbm.at[pl.ds(((i * B + j) * T) * E + base, CH)]
                    src = b0.at[s] if j == 0 else b1.at[s]
                    cps.append(pltpu.async_copy(src, dst, sem_out[s]))
            return cps

        def compute(c):
            s = c % 2

            def vec_body(i, carry):
                off = i * (_L * _U)
                for u in range(_U):
                    sl = pl.ds(off + u * _L, _L)
                    wv = bw[s, sl]
                    b0[s, sl] = b0[s, sl] + wv
                    b1[s, sl] = b1[s, sl] + wv
                return carry

            lax.fori_loop(0, CH // (_L * _U), vec_body, 0)

        in_cps = {0: start_in(0)}
        out_cps = {}
        for c in range(n_chunks):
            for cp in in_cps.pop(c):
                cp.wait()
            if c + 1 < n_chunks:
                for cp in out_cps.pop(c - 1, ()):
                    cp.wait()
                in_cps[c + 1] = start_in(c + 1)
            compute(c)
            out_cps[c] = start_out(c)
        for c, cps in sorted(out_cps.items()):
            for cp in cps:
                cp.wait()

    out = sc_add(x.reshape(-1), embed_weight.reshape(-1))
    return out.reshape(B, B, T, E)


# speedup vs baseline: 1.5550x; 1.5457x over previous
import functools

import jax
import jax.numpy as jnp
from jax import lax
from jax.experimental import pallas as pl
from jax.experimental.pallas import tpu as pltpu
from jax.experimental.pallas import tpu_sc as plsc

_NC, _NS, _L = 2, 16, 16
_NW = _NC * _NS
_R = 16
_U = 8


def kernel(x, embed_weight):
    B, T, E = x.shape
    rows_w = T // _NW
    n_chunks = rows_w // _R

    mesh = plsc.VectorSubcoreMesh(core_axis_name="c", subcore_axis_name="s")

    @functools.partial(
        pl.kernel,
        mesh=mesh,
        out_type=jax.ShapeDtypeStruct((B, B, T, E), jnp.float32),
        scratch_types=[
            pltpu.VMEM((2, _R, E), jnp.float32),
            pltpu.VMEM((2, _R, E), jnp.float32),
            pltpu.VMEM((2, _R, E), jnp.float32),
            pltpu.SemaphoreType.DMA,
            pltpu.SemaphoreType.DMA,
            pltpu.SemaphoreType.DMA,
            pltpu.SemaphoreType.DMA,
        ],
    )
    def sc_add(x_hbm, w_hbm, out_hbm, b0, b1, bw, si0, si1, so0, so1):
        wid = lax.axis_index("s") * _NC + lax.axis_index("c")
        row0 = wid * rows_w
        sem_in = (si0, si1)
        sem_out = (so0, so1)

        def start_in(c):
            s = c % 2
            rows = pl.ds(row0 + c * _R, _R)
            return [
                pltpu.async_copy(x_hbm.at[0, rows], b0.at[s], sem_in[s]),
                pltpu.async_copy(x_hbm.at[1, rows], b1.at[s], sem_in[s]),
                pltpu.async_copy(w_hbm.at[rows], bw.at[s], sem_in[s]),
            ]

        def start_out(c):
            s = c % 2
            rows = pl.ds(row0 + c * _R, _R)
            cps = []
            for i in range(B):
                for j in range(B):
                    src = b0.at[s] if j == 0 else b1.at[s]
                    cps.append(
                        pltpu.async_copy(src, out_hbm.at[i, j, rows], sem_out[s]))
            return cps

        def compute(c):
            s = c % 2

            def row_body(r, carry):
                def vec_body(i, carry2):
                    off = i * (_L * _U)
                    for u in range(_U):
                        sl = pl.ds(off + u * _L, _L)
                        wv = bw[s, r, sl]
                        b0[s, r, sl] = b0[s, r, sl] + wv
                        b1[s, r, sl] = b1[s, r, sl] + wv
                    return carry2

                return lax.fori_loop(0, E // (_L * _U), vec_body, carry)

            lax.fori_loop(0, _R, row_body, 0)

        in_cps = {0: start_in(0)}
        out_cps = {}
        for c in range(n_chunks):
            for cp in in_cps.pop(c):
                cp.wait()
            if c + 1 < n_chunks:
                for cp in out_cps.pop(c - 1, ()):
                    cp.wait()
                in_cps[c + 1] = start_in(c + 1)
            compute(c)
            out_cps[c] = start_out(c)
        for c, cps in sorted(out_cps.items()):
            for cp in cps:
                cp.wait()

    return sc_add(x, embed_weight)
